# Initial kernel scaffold; baseline (speedup 1.0000x reference)
import jax
import jax.numpy as jnp
from jax.experimental import pallas as pl


def kernel(x, weight, expert_biases):
    raise NotImplementedError("write your pallas kernel here")


# speedup vs baseline: 1.4364x; 1.4364x over previous
import jax
import jax.numpy as jnp
from jax.experimental import pallas as pl

DIM = 4096
N_EXPERTS = 64
TOPK = 8
BLK = 512


def _gate_block(x_ref, wt_ref, b_ref, ow_ref, oi_ref):
    x = x_ref[...]
    wt = wt_ref[...]
    logits = jax.lax.dot_general(
        x, wt, (((1,), (0,)), ((), ())),
        preferred_element_type=jnp.float32,
        precision=jax.lax.Precision.DEFAULT,
    )
    scores = jax.nn.sigmoid(logits)
    routing = scores + b_ref[...]
    col = jax.lax.broadcasted_iota(jnp.int32, (BLK, N_EXPERTS), 1)
    r = routing
    vals = []
    idxs = []
    for _ in range(TOPK):
        m = jnp.max(r, axis=1, keepdims=True)
        idx = jnp.min(jnp.where(r == m, col, N_EXPERTS), axis=1, keepdims=True)
        hit = col == idx
        vals.append(jnp.sum(jnp.where(hit, scores, 0.0), axis=1, keepdims=True))
        idxs.append(idx)
        r = jnp.where(hit, -jnp.inf, r)
    w = jnp.concatenate(vals, axis=1)
    i = jnp.concatenate(idxs, axis=1)
    w = w / jnp.sum(w, axis=1, keepdims=True)
    ow_ref[...] = w
    oi_ref[...] = i


def kernel(x, weight, expert_biases):
    n_tokens = x.shape[0]
    wt = weight.T
    b = expert_biases.reshape(1, N_EXPERTS)
    grid = (n_tokens // BLK,)
    ow, oi = pl.pallas_call(
        _gate_block,
        grid=grid,
        in_specs=[
            pl.BlockSpec((BLK, DIM), lambda i: (i, 0)),
            pl.BlockSpec((DIM, N_EXPERTS), lambda i: (0, 0)),
            pl.BlockSpec((1, N_EXPERTS), lambda i: (0, 0)),
        ],
        out_specs=[
            pl.BlockSpec((BLK, TOPK), lambda i: (i, 0)),
            pl.BlockSpec((BLK, TOPK), lambda i: (i, 0)),
        ],
        out_shape=[
            jax.ShapeDtypeStruct((n_tokens, TOPK), jnp.float32),
            jax.ShapeDtypeStruct((n_tokens, TOPK), jnp.int32),
        ],
    )(x, wt, b)
    return ow.astype(x.dtype), oi


# speedup vs baseline: 1.5787x; 1.0991x over previous
import jax
import jax.numpy as jnp
from jax.experimental import pallas as pl

DIM = 4096
N_EXPERTS = 64
TOPK = 8
BLK = 512


def _gate_block(x_ref, wt_ref, b_ref, ow_ref, oi_ref):
    x = x_ref[...]
    wt = wt_ref[...]
    logits = jax.lax.dot_general(
        x, wt, (((1,), (0,)), ((), ())),
        preferred_element_type=jnp.float32,
        precision=jax.lax.Precision.DEFAULT,
    )
    scores = jax.nn.sigmoid(logits)
    routing = scores + b_ref[...]
    col = jax.lax.broadcasted_iota(jnp.int32, (BLK, N_EXPERTS), 1)
    v = routing * 0.25 + 1.5
    mant = jax.lax.bitcast_convert_type(v, jnp.int32) & 0x7FFFFF
    key = (mant << 6) | (63 - col)
    vals = []
    idxs = []
    for _ in range(TOPK):
        kmax = jnp.max(key, axis=1, keepdims=True)
        hit = key == kmax
        vals.append(jnp.sum(jnp.where(hit, scores, 0.0), axis=1, keepdims=True))
        idxs.append(63 - (kmax & 63))
        key = jnp.where(hit, 0, key)
    w = jnp.concatenate(vals, axis=1)
    i = jnp.concatenate(idxs, axis=1)
    w = w / jnp.sum(w, axis=1, keepdims=True)
    ow_ref[...] = w
    oi_ref[...] = i


def kernel(x, weight, expert_biases):
    n_tokens = x.shape[0]
    wt = weight.T
    b = expert_biases.reshape(1, N_EXPERTS)
    grid = (n_tokens // BLK,)
    ow, oi = pl.pallas_call(
        _gate_block,
        grid=grid,
        in_specs=[
            pl.BlockSpec((BLK, DIM), lambda i: (i, 0)),
            pl.BlockSpec((DIM, N_EXPERTS), lambda i: (0, 0)),
            pl.BlockSpec((1, N_EXPERTS), lambda i: (0, 0)),
        ],
        out_specs=[
            pl.BlockSpec((BLK, TOPK), lambda i: (i, 0)),
            pl.BlockSpec((BLK, TOPK), lambda i: (i, 0)),
        ],
        out_shape=[
            jax.ShapeDtypeStruct((n_tokens, TOPK), jnp.float32),
            jax.ShapeDtypeStruct((n_tokens, TOPK), jnp.int32),
        ],
    )(x, wt, b)
    return ow.astype(x.dtype), oi


# speedup vs baseline: 1.5837x; 1.0032x over previous
import jax
import jax.numpy as jnp
from jax.experimental import pallas as pl

DIM = 4096
N_EXPERTS = 64
TOPK = 8
BLK = 512


def _gate_block(x_ref, wt_ref, b_ref, ow_ref, oi_ref):
    x = x_ref[...]
    wt = wt_ref[...]
    logits = jax.lax.dot_general(
        x, wt, (((1,), (0,)), ((), ())),
        preferred_element_type=jnp.float32,
        precision=jax.lax.Precision.DEFAULT,
    )
    scores = jax.nn.sigmoid(logits)
    routing = scores + b_ref[...]
    colf = jax.lax.broadcasted_iota(jnp.int32, (BLK, N_EXPERTS), 1).astype(jnp.float32)
    r = routing
    vals = []
    idxs = []
    for _ in range(TOPK):
        m = jnp.max(r, axis=1, keepdims=True)
        idx = jnp.min(jnp.where(r == m, colf, 64.0), axis=1, keepdims=True)
        hit = colf == idx
        vals.append(jnp.sum(jnp.where(hit, scores, 0.0), axis=1, keepdims=True))
        idxs.append(idx)
        r = jnp.where(hit, -3.0, r)
    w = jnp.concatenate(vals, axis=1)
    i = jnp.concatenate(idxs, axis=1).astype(jnp.int32)
    w = w / jnp.sum(w, axis=1, keepdims=True)
    ow_ref[...] = w
    oi_ref[...] = i


def kernel(x, weight, expert_biases):
    n_tokens = x.shape[0]
    wt = weight.T
    b = expert_biases.reshape(1, N_EXPERTS)
    grid = (n_tokens // BLK,)
    ow, oi = pl.pallas_call(
        _gate_block,
        grid=grid,
        in_specs=[
            pl.BlockSpec((BLK, DIM), lambda i: (i, 0)),
            pl.BlockSpec((DIM, N_EXPERTS), lambda i: (0, 0)),
            pl.BlockSpec((1, N_EXPERTS), lambda i: (0, 0)),
        ],
        out_specs=[
            pl.BlockSpec((BLK, TOPK), lambda i: (i, 0)),
            pl.BlockSpec((BLK, TOPK), lambda i: (i, 0)),
        ],
        out_shape=[
            jax.ShapeDtypeStruct((n_tokens, TOPK), jnp.float32),
            jax.ShapeDtypeStruct((n_tokens, TOPK), jnp.int32),
        ],
    )(x, wt, b)
    return ow.astype(x.dtype), oi


# speedup vs baseline: 1.7242x; 1.0887x over previous
import jax
import jax.numpy as jnp
from jax.experimental import pallas as pl

DIM = 4096
N_EXPERTS = 64
TOPK = 8
BLK = 1024


def _gate_block(x_ref, wt_ref, b_ref, ow_ref, oi_ref):
    x = x_ref[...]
    wt = wt_ref[...]
    logits = jax.lax.dot_general(
        x, wt, (((1,), (0,)), ((), ())),
        preferred_element_type=jnp.float32,
        precision=jax.lax.Precision.DEFAULT,
    )
    scores = jax.nn.sigmoid(logits)
    routing = scores + b_ref[...]
    colf = jax.lax.broadcasted_iota(jnp.int32, (BLK, N_EXPERTS), 1).astype(jnp.float32)
    r = routing
    vals = []
    idxs = []
    for _ in range(TOPK):
        m = jnp.max(r, axis=1, keepdims=True)
        idx = jnp.min(jnp.where(r == m, colf, 64.0), axis=1, keepdims=True)
        hit = colf == idx
        vals.append(jnp.sum(jnp.where(hit, scores, 0.0), axis=1, keepdims=True))
        idxs.append(idx)
        r = jnp.where(hit, -3.0, r)
    w = jnp.concatenate(vals, axis=1)
    i = jnp.concatenate(idxs, axis=1).astype(jnp.int32)
    w = w / jnp.sum(w, axis=1, keepdims=True)
    ow_ref[...] = w
    oi_ref[...] = i


def kernel(x, weight, expert_biases):
    n_tokens = x.shape[0]
    wt = weight.T
    b = expert_biases.reshape(1, N_EXPERTS)
    grid = (n_tokens // BLK,)
    ow, oi = pl.pallas_call(
        _gate_block,
        grid=grid,
        in_specs=[
            pl.BlockSpec((BLK, DIM), lambda i: (i, 0)),
            pl.BlockSpec((DIM, N_EXPERTS), lambda i: (0, 0)),
            pl.BlockSpec((1, N_EXPERTS), lambda i: (0, 0)),
        ],
        out_specs=[
            pl.BlockSpec((BLK, TOPK), lambda i: (i, 0)),
            pl.BlockSpec((BLK, TOPK), lambda i: (i, 0)),
        ],
        out_shape=[
            jax.ShapeDtypeStruct((n_tokens, TOPK), jnp.float32),
            jax.ShapeDtypeStruct((n_tokens, TOPK), jnp.int32),
        ],
    )(x, wt, b)
    return ow.astype(x.dtype), oi
